# Initial kernel scaffold; baseline (speedup 1.0000x reference)
import jax
import jax.numpy as jnp
from jax.experimental import pallas as pl


def kernel(embedding_matrix, x, W1, b1, W2, b2):
    raise NotImplementedError("write your pallas kernel here")


# speedup vs baseline: 9.4757x; 9.4757x over previous
import jax
import jax.numpy as jnp
from jax import lax
from jax.experimental import pallas as pl
from jax.experimental.pallas import tpu as pltpu
from jax.experimental.pallas import tpu_sc as plsc

VOCAB = 100000
EMB_DIM = 128
HIDDEN = 128
BATCH = 4096
HIST = 200

NC = 2
NS = 16
NW = NC * NS
BPW = BATCH // NW
C0 = 128
C1 = HIST - C0

def _cvt_body(t_ref, o_ref):
    o_ref[...] = t_ref[...].astype(jnp.bfloat16)


_cvt = pl.pallas_call(
    _cvt_body,
    grid=(50,),
    in_specs=[pl.BlockSpec((VOCAB // 50, EMB_DIM), lambda i: (i, 0))],
    out_specs=pl.BlockSpec((VOCAB // 50, EMB_DIM), lambda i: (i, 0)),
    out_shape=jax.ShapeDtypeStruct((VOCAB, EMB_DIM), jnp.bfloat16),
)


def _pool_body(table_hbm, x_hbm, out_hbm, idx_v, rows_v, pooled_v, sem0, sem1):
    wid = lax.axis_index("s") * NC + lax.axis_index("c")
    base = wid * BPW
    pltpu.sync_copy(x_hbm.at[pl.ds(base, BPW)], idx_v)
    sems = (sem0, sem1)

    def issue(b, buf):
        pltpu.async_copy(
            table_hbm.at[idx_v.at[b, pl.ds(0, C0)]],
            rows_v.at[buf, pl.ds(0, C0)], sems[buf])
        pltpu.async_copy(
            table_hbm.at[idx_v.at[b, pl.ds(C0, C1)]],
            rows_v.at[buf, pl.ds(C0, C1)], sems[buf])

    def wait(buf):
        pltpu.make_async_copy(
            table_hbm.at[pl.ds(0, HIST)], rows_v.at[buf], sems[buf]).wait()

    def accum(buf, out_b):
        def rbody(r, acc):
            acc = list(acc)
            for c in range(EMB_DIM // 16):
                w = rows_v[buf, r, pl.ds(c * 16, 16)]
                acc[c] = acc[c] + w.astype(jnp.float32)
            return tuple(acc)

        acc = lax.fori_loop(
            0, HIST, rbody,
            tuple(jnp.zeros((16,), jnp.float32)
                  for _ in range(EMB_DIM // 16)),
            unroll=4)
        for c in range(EMB_DIM // 16):
            pooled_v[out_b, pl.ds(c * 16, 16)] = acc[c]

    issue(0, 0)

    @pl.loop(0, BPW, step=2)
    def _loop(b):
        issue(b + 1, 1)
        wait(0)
        accum(0, b)

        @pl.when(b + 2 < BPW)
        def _():
            issue(b + 2, 0)

        wait(1)
        accum(1, b + 1)

    pltpu.sync_copy(pooled_v, out_hbm.at[pl.ds(base, BPW)])


_pool = pl.kernel(
    _pool_body,
    out_type=jax.ShapeDtypeStruct((BATCH, EMB_DIM), jnp.float32),
    compiler_params=pltpu.CompilerParams(use_tc_tiling_on_sc=False),
    mesh=plsc.VectorSubcoreMesh(core_axis_name="c", subcore_axis_name="s"),
    scratch_types=[
        pltpu.VMEM((BPW, HIST), jnp.int32),
        pltpu.VMEM((2, HIST, EMB_DIM), jnp.bfloat16),
        pltpu.VMEM((BPW, EMB_DIM), jnp.float32),
        pltpu.SemaphoreType.DMA,
        pltpu.SemaphoreType.DMA,
    ],
)


def _mlp_body(p_ref, w1_ref, b1_ref, w2_ref, b2_ref, o_ref):
    avg = p_ref[...] * (1.0 / HIST)
    h = jnp.maximum(
        jnp.dot(avg, w1_ref[...], preferred_element_type=jnp.float32)
        + b1_ref[...], 0.0)
    o_ref[...] = (
        jnp.dot(h, w2_ref[...], preferred_element_type=jnp.float32)
        + b2_ref[0])


@jax.jit
def kernel(embedding_matrix, x, W1, b1, W2, b2):
    x32 = x.astype(jnp.int32)
    table16 = _cvt(embedding_matrix)
    pooled = _pool(table16, x32)
    out = pl.pallas_call(
        _mlp_body,
        out_shape=jax.ShapeDtypeStruct((BATCH, 1), jnp.float32),
        in_specs=[
            pl.BlockSpec(memory_space=pltpu.VMEM),
            pl.BlockSpec(memory_space=pltpu.VMEM),
            pl.BlockSpec(memory_space=pltpu.VMEM),
            pl.BlockSpec(memory_space=pltpu.VMEM),
            pl.BlockSpec(memory_space=pltpu.SMEM),
        ],
        out_specs=pl.BlockSpec(memory_space=pltpu.VMEM),
    )(pooled, W1, b1, W2, b2)
    return out.reshape(BATCH)


# speedup vs baseline: 9.4763x; 1.0001x over previous
import jax
import jax.numpy as jnp
from jax import lax
from jax.experimental import pallas as pl
from jax.experimental.pallas import tpu as pltpu
from jax.experimental.pallas import tpu_sc as plsc

VOCAB = 100000
EMB_DIM = 128
HIDDEN = 128
BATCH = 4096
HIST = 200

NC = 2
NS = 16
NW = NC * NS
BPW = BATCH // NW
C0 = 128
C1 = HIST - C0

def _cvt_body(t_ref, o_ref):
    o_ref[...] = t_ref[...].astype(jnp.bfloat16).reshape(-1)


_cvt = pl.pallas_call(
    _cvt_body,
    grid=(50,),
    in_specs=[pl.BlockSpec((VOCAB // 50, EMB_DIM), lambda i: (i, 0))],
    out_specs=pl.BlockSpec((VOCAB // 50 * EMB_DIM,), lambda i: (i,)),
    out_shape=jax.ShapeDtypeStruct((VOCAB * EMB_DIM,), jnp.bfloat16),
)


def _pool_body(table_hbm, x_hbm, out_hbm, idx_v, rows_v, pooled_v, sem0, sem1):
    wid = lax.axis_index("s") * NC + lax.axis_index("c")
    base = wid * BPW
    pltpu.sync_copy(x_hbm.at[pl.ds(base, BPW)], idx_v)
    sems = (sem0, sem1)

    def issue(b, buf):
        pltpu.async_copy(
            table_hbm.at[idx_v.at[b, pl.ds(0, C0)]],
            rows_v.at[buf, pl.ds(0, C0)], sems[buf])
        pltpu.async_copy(
            table_hbm.at[idx_v.at[b, pl.ds(C0, C1)]],
            rows_v.at[buf, pl.ds(C0, C1)], sems[buf])

    def wait(buf):
        pltpu.make_async_copy(
            table_hbm.at[pl.ds(0, HIST)], rows_v.at[buf], sems[buf]).wait()

    def accum(buf, out_b):
        def rbody(r, acc):
            acc = list(acc)
            for c in range(EMB_DIM // 16):
                w = rows_v[buf, r, pl.ds(c * 16, 16)]
                acc[c] = acc[c] + w.astype(jnp.float32)
            return tuple(acc)

        acc = lax.fori_loop(
            0, HIST, rbody,
            tuple(jnp.zeros((16,), jnp.float32)
                  for _ in range(EMB_DIM // 16)),
            unroll=4)
        for c in range(EMB_DIM // 16):
            pooled_v[out_b, pl.ds(c * 16, 16)] = acc[c]

    issue(0, 0)

    @pl.loop(0, BPW, step=2)
    def _loop(b):
        issue(b + 1, 1)
        wait(0)
        accum(0, b)

        @pl.when(b + 2 < BPW)
        def _():
            issue(b + 2, 0)

        wait(1)
        accum(1, b + 1)

    pltpu.sync_copy(pooled_v, out_hbm.at[pl.ds(base, BPW)])


_pool = pl.kernel(
    _pool_body,
    out_type=jax.ShapeDtypeStruct((BATCH, EMB_DIM), jnp.float32),
    compiler_params=pltpu.CompilerParams(use_tc_tiling_on_sc=False),
    mesh=plsc.VectorSubcoreMesh(core_axis_name="c", subcore_axis_name="s"),
    scratch_types=[
        pltpu.VMEM((BPW, HIST), jnp.int32),
        pltpu.VMEM((2, HIST, EMB_DIM), jnp.bfloat16),
        pltpu.VMEM((BPW, EMB_DIM), jnp.float32),
        pltpu.SemaphoreType.DMA,
        pltpu.SemaphoreType.DMA,
    ],
)


def _mlp_body(p_ref, w1_ref, b1_ref, w2_ref, b2_ref, o_ref):
    avg = p_ref[...] * (1.0 / HIST)
    h = jnp.maximum(
        jnp.dot(avg, w1_ref[...], preferred_element_type=jnp.float32)
        + b1_ref[...], 0.0)
    o_ref[...] = (
        jnp.dot(h, w2_ref[...], preferred_element_type=jnp.float32)
        + b2_ref[0])


@jax.jit
def kernel(embedding_matrix, x, W1, b1, W2, b2):
    x32 = x.astype(jnp.int32)
    table16 = _cvt(embedding_matrix).reshape(VOCAB, EMB_DIM)
    pooled = _pool(table16, x32)
    out = pl.pallas_call(
        _mlp_body,
        out_shape=jax.ShapeDtypeStruct((BATCH, 1), jnp.float32),
        in_specs=[
            pl.BlockSpec(memory_space=pltpu.VMEM),
            pl.BlockSpec(memory_space=pltpu.VMEM),
            pl.BlockSpec(memory_space=pltpu.VMEM),
            pl.BlockSpec(memory_space=pltpu.VMEM),
            pl.BlockSpec(memory_space=pltpu.SMEM),
        ],
        out_specs=pl.BlockSpec(memory_space=pltpu.VMEM),
    )(pooled, W1, b1, W2, b2)
    return out.reshape(BATCH)


# speedup vs baseline: 10.0218x; 1.0576x over previous
import jax
import jax.numpy as jnp
from jax import lax
from jax.experimental import pallas as pl
from jax.experimental.pallas import tpu as pltpu
from jax.experimental.pallas import tpu_sc as plsc

VOCAB = 100000
EMB_DIM = 128
HIDDEN = 128
BATCH = 4096
HIST = 200

NC = 2
NS = 16
NW = NC * NS
BPW = BATCH // NW
C0 = 128
C1 = HIST - C0

RPW = VOCAB // NW
CCH = 125


def _cvt_body(t_hbm, o_hbm, in_v, out_v, si0, si1, so0, so1):
    wid = lax.axis_index("s") * NC + lax.axis_index("c")
    base = wid * RPW
    sis = (si0, si1)
    sos = (so0, so1)

    def issue_in(c, buf):
        pltpu.async_copy(
            t_hbm.at[pl.ds(base + c * CCH, CCH)], in_v.at[buf], sis[buf])

    def conv(buf):
        def rbody(r, carry):
            for g in range(EMB_DIM // 16):
                out_v[buf, r, pl.ds(g * 16, 16)] = (
                    in_v[buf, r, pl.ds(g * 16, 16)].astype(jnp.bfloat16))
            return carry

        lax.fori_loop(0, CCH, rbody, 0, unroll=4)

    NCH = RPW // CCH

    def wait_in(buf):
        pltpu.make_async_copy(
            t_hbm.at[pl.ds(0, CCH)], in_v.at[buf], sis[buf]).wait()

    def wait_out(buf):
        pltpu.make_async_copy(
            out_v.at[buf], o_hbm.at[pl.ds(0, CCH)], sos[buf]).wait()

    def issue_out(c, buf):
        pltpu.async_copy(
            out_v.at[buf], o_hbm.at[pl.ds(base + c * CCH, CCH)], sos[buf])

    issue_in(0, 0)

    @pl.loop(0, NCH, step=2)
    def _chunks(c):
        wait_in(0)

        @pl.when(c + 1 < NCH)
        def _():
            issue_in(c + 1, 1)

        @pl.when(c >= 2)
        def _():
            wait_out(0)

        conv(0)
        issue_out(c, 0)

        @pl.when(c + 1 < NCH)
        def _buf1():
            wait_in(1)

            @pl.when(c + 2 < NCH)
            def _():
                issue_in(c + 2, 0)

            @pl.when(c >= 1)
            def _():
                wait_out(1)

            conv(1)
            issue_out(c + 1, 1)

    wait_out(0)
    wait_out(1)


_cvt = pl.kernel(
    _cvt_body,
    out_type=jax.ShapeDtypeStruct((VOCAB, EMB_DIM), jnp.bfloat16),
    compiler_params=pltpu.CompilerParams(use_tc_tiling_on_sc=False),
    mesh=plsc.VectorSubcoreMesh(core_axis_name="c", subcore_axis_name="s"),
    scratch_types=[
        pltpu.VMEM((2, CCH, EMB_DIM), jnp.float32),
        pltpu.VMEM((2, CCH, EMB_DIM), jnp.bfloat16),
        pltpu.SemaphoreType.DMA,
        pltpu.SemaphoreType.DMA,
        pltpu.SemaphoreType.DMA,
        pltpu.SemaphoreType.DMA,
    ],
)


def _pool_body(table_hbm, x_hbm, out_hbm, idx_v, rows_v, pooled_v, sem0, sem1):
    wid = lax.axis_index("s") * NC + lax.axis_index("c")
    base = wid * BPW
    pltpu.sync_copy(x_hbm.at[pl.ds(base, BPW)], idx_v)
    sems = (sem0, sem1)

    def issue(b, buf):
        pltpu.async_copy(
            table_hbm.at[idx_v.at[b, pl.ds(0, C0)]],
            rows_v.at[buf, pl.ds(0, C0)], sems[buf])
        pltpu.async_copy(
            table_hbm.at[idx_v.at[b, pl.ds(C0, C1)]],
            rows_v.at[buf, pl.ds(C0, C1)], sems[buf])

    def wait(buf):
        pltpu.make_async_copy(
            table_hbm.at[pl.ds(0, HIST)], rows_v.at[buf], sems[buf]).wait()

    def accum(buf, out_b):
        def rbody(r, acc):
            acc = list(acc)
            for c in range(EMB_DIM // 16):
                w = rows_v[buf, r, pl.ds(c * 16, 16)]
                acc[c] = acc[c] + w.astype(jnp.float32)
            return tuple(acc)

        acc = lax.fori_loop(
            0, HIST, rbody,
            tuple(jnp.zeros((16,), jnp.float32)
                  for _ in range(EMB_DIM // 16)),
            unroll=4)
        for c in range(EMB_DIM // 16):
            pooled_v[out_b, pl.ds(c * 16, 16)] = acc[c]

    issue(0, 0)

    @pl.loop(0, BPW, step=2)
    def _loop(b):
        issue(b + 1, 1)
        wait(0)
        accum(0, b)

        @pl.when(b + 2 < BPW)
        def _():
            issue(b + 2, 0)

        wait(1)
        accum(1, b + 1)

    pltpu.sync_copy(pooled_v, out_hbm.at[pl.ds(base, BPW)])


_pool = pl.kernel(
    _pool_body,
    out_type=jax.ShapeDtypeStruct((BATCH, EMB_DIM), jnp.float32),
    compiler_params=pltpu.CompilerParams(use_tc_tiling_on_sc=False),
    mesh=plsc.VectorSubcoreMesh(core_axis_name="c", subcore_axis_name="s"),
    scratch_types=[
        pltpu.VMEM((BPW, HIST), jnp.int32),
        pltpu.VMEM((2, HIST, EMB_DIM), jnp.bfloat16),
        pltpu.VMEM((BPW, EMB_DIM), jnp.float32),
        pltpu.SemaphoreType.DMA,
        pltpu.SemaphoreType.DMA,
    ],
)


def _mlp_body(p_ref, w1_ref, b1_ref, w2_ref, b2_ref, o_ref):
    avg = p_ref[...] * (1.0 / HIST)
    h = jnp.maximum(
        jnp.dot(avg, w1_ref[...], preferred_element_type=jnp.float32)
        + b1_ref[...], 0.0)
    o_ref[...] = (
        jnp.dot(h, w2_ref[...], preferred_element_type=jnp.float32)
        + b2_ref[0])


@jax.jit
def kernel(embedding_matrix, x, W1, b1, W2, b2):
    x32 = x.astype(jnp.int32)
    table16 = _cvt(embedding_matrix)
    pooled = _pool(table16, x32)
    out = pl.pallas_call(
        _mlp_body,
        out_shape=jax.ShapeDtypeStruct((BATCH, 1), jnp.float32),
        in_specs=[
            pl.BlockSpec(memory_space=pltpu.VMEM),
            pl.BlockSpec(memory_space=pltpu.VMEM),
            pl.BlockSpec(memory_space=pltpu.VMEM),
            pl.BlockSpec(memory_space=pltpu.VMEM),
            pl.BlockSpec(memory_space=pltpu.SMEM),
        ],
        out_specs=pl.BlockSpec(memory_space=pltpu.VMEM),
    )(pooled, W1, b1, W2, b2)
    return out.reshape(BATCH)
